# baseline (speedup 1.0000x reference)
import jax
import jax.numpy as jnp
from jax import lax
from jax.experimental import pallas as pl
from jax.experimental.pallas import tpu as pltpu
from jax.experimental.pallas import tpu_sc as plsc

_NB = 4
_NC = 768
_D = 8192
_ROWS = _NB * _NC
_NW = 32
_RPW = _ROWS // _NW


def _body(in_hbm, out_hbm, sem):
    cid = lax.axis_index("c")
    sid = lax.axis_index("s")
    wid = cid * 16 + sid
    base = wid * _RPW
    b = base // _NC
    src_hi = 2 * b * _NC + (_NC - 1) - base

    cps = []
    for j in range(_RPW):
        cps.append(pltpu.async_copy(
            in_hbm.at[pl.ds((src_hi - j) * _D, _D)],
            out_hbm.at[pl.ds((base + j) * _D, _D)], sem))
    for cp in cps:
        cp.wait()


@jax.jit
def _permute(x1d):
    mesh = plsc.VectorSubcoreMesh(core_axis_name="c", subcore_axis_name="s")
    return pl.kernel(
        _body,
        mesh=mesh,
        out_type=jax.ShapeDtypeStruct((_ROWS * _D,), jnp.float32),
        scratch_types=[
            pltpu.SemaphoreType.DMA,
        ],
    )(x1d)


def kernel(input):
    x1d = input.reshape(_ROWS * _D)
    out = _permute(x1d)
    return out.reshape(_NB, _NC, _D)


# speedup vs baseline: 10.6785x; 10.6785x over previous
import jax
import jax.numpy as jnp
from jax import lax
from jax.experimental import pallas as pl
from jax.experimental.pallas import tpu as pltpu
from jax.experimental.pallas import tpu_sc as plsc

_NB = 4
_NC = 768
_D = 8192
_SPLIT = 4
_QD = _D // _SPLIT
_QROWS = _NB * _NC * _SPLIT
_NW = 32
_QPW = _QROWS // _NW
_K = 16
_NCHUNK = _QPW // _K
_NSLOT = 3


def _body(in_hbm, out_hbm,
          idx0, idx1, idx2, buf0, buf1, buf2, sem_g, sem_s):
    cid = lax.axis_index("c")
    sid = lax.axis_index("s")
    wid = cid * 16 + sid
    base_q = wid * _QPW
    r0 = base_q // _SPLIT
    b = r0 // _NC
    s0 = (2 * b * _NC + _NC - 1) - r0

    iv = lax.iota(jnp.int32, 16)
    v0 = (4 * s0) - (iv & ~3) + (iv & 3)

    idxs = (idx0, idx1, idx2)
    bufs = (buf0, buf1, buf2)

    def start_gather(g, slot):
        idxs[slot][...] = v0 - 16 * g
        return pltpu.async_copy(in_hbm.at[idxs[slot]], bufs[slot], sem_g)

    def start_store(g, slot):
        return pltpu.async_copy(
            bufs[slot], out_hbm.at[pl.ds(base_q + g * _K, _K)], sem_s)

    gath = {}
    for g in range(_NSLOT):
        gath[g] = start_gather(g, g % _NSLOT)
    st = {}
    for g in range(_NCHUNK):
        slot = g % _NSLOT
        gath[g].wait()
        st[g] = start_store(g, slot)
        if g + _NSLOT < _NCHUNK:
            st[g].wait()
            gath[g + _NSLOT] = start_gather(g + _NSLOT, slot)
    for g in range(_NCHUNK - _NSLOT, _NCHUNK):
        st[g].wait()


@jax.jit
def _permute(x2d):
    mesh = plsc.VectorSubcoreMesh(core_axis_name="c", subcore_axis_name="s")
    return pl.kernel(
        _body,
        mesh=mesh,
        out_type=jax.ShapeDtypeStruct((_QROWS, _QD), jnp.float32),
        scratch_types=[
            pltpu.VMEM((_K,), jnp.int32),
            pltpu.VMEM((_K,), jnp.int32),
            pltpu.VMEM((_K,), jnp.int32),
            pltpu.VMEM((_K, _QD), jnp.float32),
            pltpu.VMEM((_K, _QD), jnp.float32),
            pltpu.VMEM((_K, _QD), jnp.float32),
            pltpu.SemaphoreType.DMA,
            pltpu.SemaphoreType.DMA,
        ],
    )(x2d)


def kernel(input):
    x2d = input.reshape(_QROWS, _QD)
    out = _permute(x2d)
    return out.reshape(_NB, _NC, _D)


# speedup vs baseline: 35.9482x; 3.3664x over previous
import jax
import jax.numpy as jnp
from jax import lax
from jax.experimental import pallas as pl
from jax.experimental.pallas import tpu as pltpu
from jax.experimental.pallas import tpu_sc as plsc

_NB = 4
_NC = 768
_D = 8192
_K = 16
_QD = 2048
_NQ = _D // _QD
_NCG = _NC // _K
_TASKS = _NB * _NCG * _NQ
_NW = 32
_TPW = _TASKS // _NW
_NSLOT = 3


def _body(in_hbm, out_hbm,
          idx0, idx1, idx2, buf0, buf1, buf2, sem_g, sem_s):
    cid = lax.axis_index("c")
    sid = lax.axis_index("s")
    wid = cid * 16 + sid
    t0 = wid * _TPW

    iv = lax.iota(jnp.int32, _K)
    idxs = (idx0, idx1, idx2)
    bufs = (buf0, buf1, buf2)

    def task(k):
        t = t0 + k
        b = t // (_NCG * _NQ)
        rem = t % (_NCG * _NQ)
        cg = rem // _NQ
        q = rem % _NQ
        return b, cg * _K, q * _QD

    def start_gather(k, slot):
        b, o0, qq = task(k)
        idxs[slot][...] = (_NC - 1 - o0) - iv
        src = in_hbm.at[b, :, pl.ds(qq, _QD)]
        return pltpu.async_copy(src.at[idxs[slot]], bufs[slot], sem_g)

    def start_store(k, slot):
        b, o0, qq = task(k)
        dst = out_hbm.at[b, pl.ds(o0, _K), pl.ds(qq, _QD)]
        return pltpu.async_copy(bufs[slot], dst, sem_s)

    gath = {}
    for k in range(_NSLOT):
        gath[k] = start_gather(k, k % _NSLOT)
    st = {}
    for k in range(_TPW):
        slot = k % _NSLOT
        gath[k].wait()
        st[k] = start_store(k, slot)
        if k + _NSLOT < _TPW:
            st[k].wait()
            gath[k + _NSLOT] = start_gather(k + _NSLOT, slot)
    for k in range(_TPW - _NSLOT, _TPW):
        st[k].wait()


@jax.jit
def _permute(x):
    mesh = plsc.VectorSubcoreMesh(core_axis_name="c", subcore_axis_name="s")
    return pl.kernel(
        _body,
        mesh=mesh,
        out_type=jax.ShapeDtypeStruct((_NB, _NC, _D), jnp.float32),
        scratch_types=[
            pltpu.VMEM((_K,), jnp.int32),
            pltpu.VMEM((_K,), jnp.int32),
            pltpu.VMEM((_K,), jnp.int32),
            pltpu.VMEM((_K, _QD), jnp.float32),
            pltpu.VMEM((_K, _QD), jnp.float32),
            pltpu.VMEM((_K, _QD), jnp.float32),
            pltpu.SemaphoreType.DMA,
            pltpu.SemaphoreType.DMA,
        ],
    )(x)


def kernel(input):
    return _permute(input)
